# baseline (speedup 1.0000x reference)
import jax
import jax.numpy as jnp
from jax import lax
from jax.experimental import pallas as pl
from jax.experimental.pallas import tpu as pltpu
from jax.experimental.pallas import tpu_sc as plsc

_NC = 2
_NS = 16
_NW = _NC * _NS
_LANES = 16
_CHUNK = 32768
_UNROLL = 8


def _sc_add_body(x_hbm, e_hbm, o_hbm, xb, eb):
    wid = lax.axis_index("s") * _NC + lax.axis_index("c")
    total = x_hbm.shape[0]
    per_w = total // _NW
    n_chunks = per_w // _CHUNK
    base = wid * per_w

    def chunk_body(ci, carry):
        off = base + ci * _CHUNK
        pltpu.sync_copy(x_hbm.at[pl.ds(off, _CHUNK)], xb)
        pltpu.sync_copy(e_hbm.at[pl.ds(off, _CHUNK)], eb)

        def add_body(i, c):
            b = i * (_LANES * _UNROLL)
            for u in range(_UNROLL):
                s = b + u * _LANES
                xb[pl.ds(s, _LANES)] = xb[pl.ds(s, _LANES)] + eb[pl.ds(s, _LANES)]
            return c

        lax.fori_loop(0, _CHUNK // (_LANES * _UNROLL), add_body, 0)
        pltpu.sync_copy(xb, o_hbm.at[pl.ds(off, _CHUNK)])
        return carry

    lax.fori_loop(0, n_chunks, chunk_body, 0)


def kernel(x, emb_weight):
    seq_len, dim = x.shape
    total = seq_len * dim
    xf = x.reshape(total)
    ef = emb_weight[:seq_len].reshape(total)
    mesh = plsc.VectorSubcoreMesh(core_axis_name="c", subcore_axis_name="s")
    sc_call = pl.kernel(
        _sc_add_body,
        out_type=jax.ShapeDtypeStruct((total,), jnp.float32),
        mesh=mesh,
        scratch_types=[
            pltpu.VMEM((_CHUNK,), jnp.float32),
            pltpu.VMEM((_CHUNK,), jnp.float32),
        ],
    )
    return sc_call(xf, ef).reshape(seq_len, dim)


# speedup vs baseline: 1.1687x; 1.1687x over previous
import jax
import jax.numpy as jnp
from jax import lax
from jax.experimental import pallas as pl
from jax.experimental.pallas import tpu as pltpu
from jax.experimental.pallas import tpu_sc as plsc

_NC = 2
_NS = 16
_NW = _NC * _NS
_LANES = 16
_CHUNK = 16384
_NBUF = 2


def _sc_add_body(x_hbm, e_hbm, o_hbm,
                 xb0, eb0, ob0, xb1, eb1, ob1,
                 sem_in0, sem_in1, sem_out0, sem_out1):
    xb = (xb0, xb1)
    eb = (eb0, eb1)
    ob = (ob0, ob1)
    sem_in = (sem_in0, sem_in1)
    sem_out = (sem_out0, sem_out1)

    wid = lax.axis_index("s") * _NC + lax.axis_index("c")
    total = x_hbm.shape[0]
    per_w = total // _NW
    n_chunks = per_w // _CHUNK
    n_outer = n_chunks // _NBUF
    base = wid * per_w

    def fire_in(b, ci):
        off = base + ci * _CHUNK
        pltpu.async_copy(x_hbm.at[pl.ds(off, _CHUNK)], xb[b], sem_in[b])
        pltpu.async_copy(e_hbm.at[pl.ds(off, _CHUNK)], eb[b], sem_in[b])

    def wait_in(b, ci):
        off = base + ci * _CHUNK
        pltpu.make_async_copy(x_hbm.at[pl.ds(off, _CHUNK)], xb[b], sem_in[b]).wait()
        pltpu.make_async_copy(e_hbm.at[pl.ds(off, _CHUNK)], eb[b], sem_in[b]).wait()

    def fire_out(b, ci):
        off = base + ci * _CHUNK
        pltpu.async_copy(ob[b], o_hbm.at[pl.ds(off, _CHUNK)], sem_out[b])

    def wait_out(b, ci):
        off = base + ci * _CHUNK
        pltpu.make_async_copy(ob[b], o_hbm.at[pl.ds(off, _CHUNK)], sem_out[b]).wait()

    for b in range(_NBUF):
        fire_in(b, b)

    def outer(g, carry):
        for b in range(_NBUF):
            ci = g * _NBUF + b
            wait_in(b, ci)

            @pl.when(g > 0)
            def _():
                wait_out(b, ci - _NBUF)

            @plsc.parallel_loop(0, _CHUNK // _LANES, 1, unroll=8)
            def _(i):
                s = i * _LANES
                ob[b][pl.ds(s, _LANES)] = (
                    xb[b][pl.ds(s, _LANES)] + eb[b][pl.ds(s, _LANES)]
                )

            fire_out(b, ci)

            @pl.when(ci + _NBUF < n_chunks)
            def _():
                fire_in(b, ci + _NBUF)

        return carry

    lax.fori_loop(0, n_outer, outer, 0)

    for b in range(_NBUF):
        wait_out(b, n_chunks - _NBUF + b)


def kernel(x, emb_weight):
    seq_len, dim = x.shape
    total = seq_len * dim
    xf = x.reshape(total)
    ef = emb_weight[:seq_len].reshape(total)
    mesh = plsc.VectorSubcoreMesh(core_axis_name="c", subcore_axis_name="s")
    sc_call = pl.kernel(
        _sc_add_body,
        out_type=jax.ShapeDtypeStruct((total,), jnp.float32),
        mesh=mesh,
        scratch_types=[
            pltpu.VMEM((_CHUNK,), jnp.float32),
            pltpu.VMEM((_CHUNK,), jnp.float32),
            pltpu.VMEM((_CHUNK,), jnp.float32),
            pltpu.VMEM((_CHUNK,), jnp.float32),
            pltpu.VMEM((_CHUNK,), jnp.float32),
            pltpu.VMEM((_CHUNK,), jnp.float32),
            pltpu.SemaphoreType.DMA,
            pltpu.SemaphoreType.DMA,
            pltpu.SemaphoreType.DMA,
            pltpu.SemaphoreType.DMA,
        ],
    )
    return sc_call(xf, ef).reshape(seq_len, dim)


# speedup vs baseline: 3.0900x; 2.6440x over previous
import jax
import jax.numpy as jnp
from jax import lax
from jax.experimental import pallas as pl
from jax.experimental.pallas import tpu as pltpu
from jax.experimental.pallas import tpu_sc as plsc

_NC = 2
_NS = 16
_NW = _NC * _NS
_LANES = 16
_ROWS = 16
_NBUF = 2


def _sc_add_body(x_hbm, e_hbm, o_hbm,
                 xb0, eb0, ob0, xb1, eb1, ob1,
                 sem_in0, sem_in1, sem_out0, sem_out1):
    xb = (xb0, xb1)
    eb = (eb0, eb1)
    ob = (ob0, ob1)
    sem_in = (sem_in0, sem_in1)
    sem_out = (sem_out0, sem_out1)

    wid = lax.axis_index("s") * _NC + lax.axis_index("c")
    rows_total = x_hbm.shape[0]
    dim = x_hbm.shape[1]
    rows_per_w = rows_total // _NW
    n_chunks = rows_per_w // _ROWS
    base = wid * rows_per_w
    vecs_per_row = dim // _LANES
    vecs_per_chunk = _ROWS * vecs_per_row

    def fire_in(b, ci):
        r0 = base + ci * _ROWS
        pltpu.async_copy(x_hbm.at[pl.ds(r0, _ROWS)], xb[b], sem_in[b])
        pltpu.async_copy(e_hbm.at[pl.ds(r0, _ROWS)], eb[b], sem_in[b])

    def wait_in(b, ci):
        r0 = base + ci * _ROWS
        pltpu.make_async_copy(x_hbm.at[pl.ds(r0, _ROWS)], xb[b], sem_in[b]).wait()
        pltpu.make_async_copy(e_hbm.at[pl.ds(r0, _ROWS)], eb[b], sem_in[b]).wait()

    def fire_out(b, ci):
        r0 = base + ci * _ROWS
        pltpu.async_copy(ob[b], o_hbm.at[pl.ds(r0, _ROWS)], sem_out[b])

    def wait_out(b, ci):
        r0 = base + ci * _ROWS
        pltpu.make_async_copy(ob[b], o_hbm.at[pl.ds(r0, _ROWS)], sem_out[b]).wait()

    for b in range(_NBUF):
        fire_in(b, b)

    def outer(g, carry):
        for b in range(_NBUF):
            ci = g * _NBUF + b
            wait_in(b, ci)

            @pl.when(g > 0)
            def _():
                wait_out(b, ci - _NBUF)

            @plsc.parallel_loop(0, vecs_per_chunk, 1, unroll=8)
            def _(i):
                r = lax.shift_right_logical(i, 6)
                c = pl.multiple_of(
                    lax.shift_left(lax.bitwise_and(i, vecs_per_row - 1), 4),
                    _LANES,
                )
                ob[b][r, pl.ds(c, _LANES)] = (
                    xb[b][r, pl.ds(c, _LANES)] + eb[b][r, pl.ds(c, _LANES)]
                )

            fire_out(b, ci)

            @pl.when(ci + _NBUF < n_chunks)
            def _():
                fire_in(b, ci + _NBUF)

        return carry

    lax.fori_loop(0, n_chunks // _NBUF, outer, 0)

    for b in range(_NBUF):
        wait_out(b, n_chunks - _NBUF + b)


def kernel(x, emb_weight):
    seq_len, dim = x.shape
    mesh = plsc.VectorSubcoreMesh(core_axis_name="c", subcore_axis_name="s")
    sc_call = pl.kernel(
        _sc_add_body,
        out_type=jax.ShapeDtypeStruct((seq_len, dim), jnp.float32),
        mesh=mesh,
        compiler_params=pltpu.CompilerParams(use_tc_tiling_on_sc=True),
        scratch_types=[
            pltpu.VMEM((_ROWS, 1024), jnp.float32),
            pltpu.VMEM((_ROWS, 1024), jnp.float32),
            pltpu.VMEM((_ROWS, 1024), jnp.float32),
            pltpu.VMEM((_ROWS, 1024), jnp.float32),
            pltpu.VMEM((_ROWS, 1024), jnp.float32),
            pltpu.VMEM((_ROWS, 1024), jnp.float32),
            pltpu.SemaphoreType.DMA,
            pltpu.SemaphoreType.DMA,
            pltpu.SemaphoreType.DMA,
            pltpu.SemaphoreType.DMA,
        ],
    )
    return sc_call(x, emb_weight[:seq_len])


# speedup vs baseline: 5.3496x; 1.7312x over previous
import jax
import jax.numpy as jnp
from jax.experimental import pallas as pl


def _add_body(x_ref, e_ref, o_ref):
    o_ref[...] = x_ref[...] + e_ref[...]


def kernel(x, emb_weight):
    seq_len, dim = x.shape
    block_rows = 1024
    grid = (seq_len // block_rows,)
    spec = pl.BlockSpec((block_rows, dim), lambda i: (i, 0))
    return pl.pallas_call(
        _add_body,
        grid=grid,
        in_specs=[spec, spec],
        out_specs=spec,
        out_shape=jax.ShapeDtypeStruct((seq_len, dim), x.dtype),
    )(x, emb_weight[:seq_len])
